# baseline (speedup 1.0000x reference)
import functools

import jax
import jax.numpy as jnp
from jax import lax
from jax.experimental import pallas as pl
from jax.experimental.pallas import tpu as pltpu
from jax.experimental.pallas import tpu_sc as plsc

VOCAB = 1000000
D = 64
WIDE = 128
LMAX = 200
B = 4096
HID = 256
NFEAT = 8
LPAD = 256
NIDX = 13


def _tc_widen(in_ref, out_ref):
    x = in_ref[...]
    out_ref[...] = jnp.concatenate(
        [x, jnp.zeros((x.shape[0], WIDE - D), jnp.float32)], axis=1)


def _widen(table):
    blk = 8000
    return pl.pallas_call(
        _tc_widen,
        grid=(VOCAB // blk,),
        in_specs=[pl.BlockSpec((blk, D), lambda i: (i, 0))],
        out_specs=pl.BlockSpec((blk, WIDE), lambda i: (i, 0)),
        out_shape=jax.ShapeDtypeStruct((VOCAB, WIDE), jnp.float32),
    )(table)


def _sc_pooled_sum():
    info = plsc.get_sparse_core_info()
    nc, ns = info.num_cores, info.num_subcores
    nw = nc * ns
    b_per_w = B // nw
    nrows = NIDX * 16
    mesh = plsc.VectorSubcoreMesh(core_axis_name="c", subcore_axis_name="s")

    @functools.partial(
        pl.kernel,
        mesh=mesh,
        compiler_params=pltpu.CompilerParams(use_tc_tiling_on_sc=False),
        out_type=jax.ShapeDtypeStruct((B, D), jnp.float32),
        scratch_types=[
            pltpu.VMEM((b_per_w, LPAD), jnp.int32),
            pltpu.VMEM((4, nrows, D), jnp.float32),
            pltpu.VMEM((b_per_w, D), jnp.float32),
            pltpu.SemaphoreType.DMA,
        ],
    )
    def k(ids_hbm, wide_hbm, out_hbm, ids_v, bufs, out_v, sem):
        wid = lax.axis_index("s") * nc + lax.axis_index("c")
        base = wid * b_per_w
        pltpu.sync_copy(ids_hbm.at[pl.ds(base, b_per_w)], ids_v)

        def issue(r, slot):
            for off, n in ((0, 128), (128, 80)):
                pltpu.async_copy(
                    wide_hbm.at[ids_v.at[r, pl.ds(off, n)]],
                    bufs.at[slot, pl.ds(off, n)], sem)

        def wait_one_row(slot):
            pltpu.make_async_copy(
                wide_hbm.at[pl.ds(0, nrows)], bufs.at[slot], sem
            ).wait()

        def accum(r, slot):
            zero = jnp.zeros((16,), jnp.float32)

            def body(l, accs):
                return tuple(
                    accs[j] + bufs[slot, l, pl.ds(j * 16, 16)]
                    for j in range(4)
                )

            accs = lax.fori_loop(0, nrows, body, (zero,) * 4, unroll=4)
            for j in range(4):
                out_v[r, pl.ds(j * 16, 16)] = accs[j]

        for r in range(3):
            issue(r, r)

        def group(g, _):
            for b in range(4):
                r = g * 4 + b
                issue(jnp.minimum(r + 3, b_per_w - 1), (b + 3) % 4)
                wait_one_row(b)
                accum(r, b)
            return 0

        lax.fori_loop(0, b_per_w // 4, group, 0)
        for _ in range(3):
            wait_one_row(0)
        pltpu.sync_copy(out_v, out_hbm.at[pl.ds(base, b_per_w)])

    return k


def _tc_head(ids_ref, pooled_ref, len_ref, tTL_ref, pos_ref, g_ref, b_ref,
             w1_ref, b1_ref, w2_ref, b2_ref, out_ref):
    f32 = jnp.float32
    m = (ids_ref[...] != 0).astype(f32)
    count = jnp.sum(m, axis=1, keepdims=True)
    pos_sum = jnp.dot(m, pos_ref[...], preferred_element_type=f32)
    denom = jnp.maximum(count, 1.0)
    seq = (pooled_ref[...] + pos_sum) / denom

    t = tTL_ref[0, 0]
    T = tTL_ref[0, 1]
    L = tTL_ref[0, 2]
    lens = len_ref[...].astype(f32)
    one = jnp.ones_like(lens)
    gap = lens - L
    rem = (T - t) * one
    prog = (t / jnp.maximum(T, 1.0)) * one
    need = gap / jnp.maximum(rem, 1.0)
    len_ratio = lens / jnp.maximum(L, 1.0)
    gap_ratio = gap / jnp.maximum(lens, 1.0)
    rem_ratio = ((T - t) / jnp.maximum(T, 1.0)) * one
    tgt_ratio = (L / jnp.maximum(T, 1.0)) * one
    feats = jnp.concatenate(
        [gap, rem, prog, need, len_ratio, gap_ratio, rem_ratio, tgt_ratio],
        axis=1)

    nf = D + NFEAT
    pad = jnp.zeros((seq.shape[0], 128 - nf), f32)
    fused = jnp.concatenate([seq, feats, pad], axis=1)
    mu = jnp.sum(fused, axis=1, keepdims=True) / nf
    var = jnp.sum(fused * fused, axis=1, keepdims=True) / nf - mu * mu
    ln = (fused - mu) * lax.rsqrt(var + 1e-5) * g_ref[...] + b_ref[...]

    h = jnp.dot(ln, w1_ref[...], preferred_element_type=f32) + b1_ref[...]
    h = 0.5 * h * (1.0 + lax.erf(h * 0.7071067811865476))
    logit = jnp.sum(h * w2_ref[...], axis=1, keepdims=True) + b2_ref[...]
    out_ref[...] = jax.nn.sigmoid(logit)


def kernel(token_ids, lengths, t, T, L, token_table, pos_table, ln_g, ln_b,
           W1, b1, W2, b2):
    f32 = jnp.float32
    ids = token_ids.astype(jnp.int32)
    ids_pad = jnp.pad(ids, ((0, 0), (0, LPAD - LMAX)))

    pooled = _sc_pooled_sum()(ids_pad, token_table)

    pos_pad = jnp.pad(pos_table[:LMAX], ((0, LPAD - LMAX), (0, 0)))
    nf = D + NFEAT
    g_pad = jnp.pad(ln_g, (0, 128 - nf)).reshape(1, 128)
    b_pad = jnp.pad(ln_b, (0, 128 - nf)).reshape(1, 128)
    w1_pad = jnp.pad(W1, ((0, 128 - nf), (0, 0)))
    tTL = jnp.stack([jnp.asarray(t, f32), jnp.asarray(T, f32),
                     jnp.asarray(L, f32)]).reshape(1, 3)

    BLK = 512
    grid = (B // BLK,)
    rep = lambda i: (0, 0)
    out = pl.pallas_call(
        _tc_head,
        grid=grid,
        in_specs=[
            pl.BlockSpec((BLK, LPAD), lambda i: (i, 0)),
            pl.BlockSpec((BLK, D), lambda i: (i, 0)),
            pl.BlockSpec((BLK, 1), lambda i: (i, 0)),
            pl.BlockSpec(memory_space=pltpu.SMEM),
            pl.BlockSpec((LPAD, D), rep),
            pl.BlockSpec((1, 128), rep),
            pl.BlockSpec((1, 128), rep),
            pl.BlockSpec((128, HID), rep),
            pl.BlockSpec((1, HID), rep),
            pl.BlockSpec((1, HID), rep),
            pl.BlockSpec((1, 1), rep),
        ],
        out_specs=pl.BlockSpec((BLK, 1), lambda i: (i, 0)),
        out_shape=jax.ShapeDtypeStruct((B, 1), f32),
    )(ids_pad, pooled, lengths.astype(jnp.int32).reshape(B, 1), tTL, pos_pad,
      g_pad, b_pad, w1_pad, b1.reshape(1, HID), W2.reshape(1, HID),
      b2.reshape(1, 1))
    return out.reshape(B)


# speedup vs baseline: 1.3198x; 1.3198x over previous
import functools

import jax
import jax.numpy as jnp
from jax import lax
from jax.experimental import pallas as pl
from jax.experimental.pallas import tpu as pltpu
from jax.experimental.pallas import tpu_sc as plsc

VOCAB = 1000000
D = 64
LMAX = 200
B = 4096
HID = 256
NFEAT = 8
LPAD = 256

CHUNKS = ((0, 128), (128, 72))


def _sc_pooled_sum():
    info = plsc.get_sparse_core_info()
    nc, ns = info.num_cores, info.num_subcores
    nw = nc * ns
    b_per_w = B // nw
    mesh = plsc.VectorSubcoreMesh(core_axis_name="c", subcore_axis_name="s")

    @functools.partial(
        pl.kernel,
        mesh=mesh,
        compiler_params=pltpu.CompilerParams(use_tc_tiling_on_sc=False),
        out_type=jax.ShapeDtypeStruct((B, D), jnp.float32),
        scratch_types=[
            pltpu.VMEM((LMAX,), jnp.int32),
            pltpu.VMEM((LMAX, D), jnp.float32),
            pltpu.VMEM((1, D), jnp.float32),
            pltpu.SemaphoreType.DMA,
        ],
    )
    def k(ids_hbm, table_hbm, out_hbm, idx_v, rows_v, acc_v, sem):
        wid = lax.axis_index("s") * nc + lax.axis_index("c")
        base = wid * b_per_w

        def body(i, _):
            b = base + i
            pltpu.sync_copy(ids_hbm.at[b], idx_v)
            for off, n in CHUNKS:
                pltpu.async_copy(
                    table_hbm.at[idx_v.at[pl.ds(off, n)]],
                    rows_v.at[pl.ds(off, n)],
                    sem,
                ).wait()
            for j in range(D // 16):
                def add(s, acc):
                    l = s * 4
                    for u in range(4):
                        acc = acc + rows_v[l + u, pl.ds(j * 16, 16)]
                    return acc
                acc = lax.fori_loop(0, LMAX // 4, add, jnp.zeros((16,), jnp.float32))
                acc_v[0, pl.ds(j * 16, 16)] = acc
            pltpu.sync_copy(acc_v, out_hbm.at[pl.ds(b, 1)])
            return 0

        lax.fori_loop(0, b_per_w, body, 0)

    return k


def _tc_head(ids_ref, pooled_ref, len_ref, tTL_ref, pos_ref, g_ref, b_ref,
             w1_ref, b1_ref, w2_ref, b2_ref, out_ref):
    f32 = jnp.float32
    m = (ids_ref[...] != 0).astype(f32)
    count = jnp.sum(m, axis=1, keepdims=True)
    pos_sum = jnp.dot(m, pos_ref[...], preferred_element_type=f32)
    denom = jnp.maximum(count, 1.0)
    seq = (pooled_ref[...] + pos_sum) / denom

    t = tTL_ref[0, 0]
    T = tTL_ref[0, 1]
    L = tTL_ref[0, 2]
    lens = len_ref[...].astype(f32)
    one = jnp.ones_like(lens)
    gap = lens - L
    rem = (T - t) * one
    prog = (t / jnp.maximum(T, 1.0)) * one
    need = gap / jnp.maximum(rem, 1.0)
    len_ratio = lens / jnp.maximum(L, 1.0)
    gap_ratio = gap / jnp.maximum(lens, 1.0)
    rem_ratio = ((T - t) / jnp.maximum(T, 1.0)) * one
    tgt_ratio = (L / jnp.maximum(T, 1.0)) * one
    feats = jnp.concatenate(
        [gap, rem, prog, need, len_ratio, gap_ratio, rem_ratio, tgt_ratio],
        axis=1)

    nf = D + NFEAT
    pad = jnp.zeros((seq.shape[0], 128 - nf), f32)
    fused = jnp.concatenate([seq, feats, pad], axis=1)
    mu = jnp.sum(fused, axis=1, keepdims=True) / nf
    var = jnp.sum(fused * fused, axis=1, keepdims=True) / nf - mu * mu
    ln = (fused - mu) * lax.rsqrt(var + 1e-5) * g_ref[...] + b_ref[...]

    h = jnp.dot(ln, w1_ref[...], preferred_element_type=f32) + b1_ref[...]
    h = 0.5 * h * (1.0 + lax.erf(h * 0.7071067811865476))
    logit = jnp.sum(h * w2_ref[...], axis=1, keepdims=True) + b2_ref[...]
    out_ref[...] = jax.nn.sigmoid(logit)


def kernel(token_ids, lengths, t, T, L, token_table, pos_table, ln_g, ln_b,
           W1, b1, W2, b2):
    f32 = jnp.float32
    ids = token_ids.astype(jnp.int32)

    pooled = _sc_pooled_sum()(ids, token_table)

    ids_pad = jnp.pad(ids, ((0, 0), (0, LPAD - LMAX)))
    pos_pad = jnp.pad(pos_table[:LMAX], ((0, LPAD - LMAX), (0, 0)))
    nf = D + NFEAT
    g_pad = jnp.pad(ln_g, (0, 128 - nf)).reshape(1, 128)
    b_pad = jnp.pad(ln_b, (0, 128 - nf)).reshape(1, 128)
    w1_pad = jnp.pad(W1, ((0, 128 - nf), (0, 0)))
    tTL = jnp.stack([jnp.asarray(t, f32), jnp.asarray(T, f32),
                     jnp.asarray(L, f32)]).reshape(1, 3)

    BLK = 512
    grid = (B // BLK,)
    rep = lambda i: (0, 0)
    out = pl.pallas_call(
        _tc_head,
        grid=grid,
        in_specs=[
            pl.BlockSpec((BLK, LPAD), lambda i: (i, 0)),
            pl.BlockSpec((BLK, D), lambda i: (i, 0)),
            pl.BlockSpec((BLK, 1), lambda i: (i, 0)),
            pl.BlockSpec(memory_space=pltpu.SMEM),
            pl.BlockSpec((LPAD, D), rep),
            pl.BlockSpec((1, 128), rep),
            pl.BlockSpec((1, 128), rep),
            pl.BlockSpec((128, HID), rep),
            pl.BlockSpec((1, HID), rep),
            pl.BlockSpec((1, HID), rep),
            pl.BlockSpec((1, 1), rep),
        ],
        out_specs=pl.BlockSpec((BLK, 1), lambda i: (i, 0)),
        out_shape=jax.ShapeDtypeStruct((B, 1), f32),
    )(ids_pad, pooled, lengths.astype(jnp.int32).reshape(B, 1), tTL, pos_pad,
      g_pad, b_pad, w1_pad, b1.reshape(1, HID), W2.reshape(1, HID),
      b2.reshape(1, 1))
    return out.reshape(B)


# speedup vs baseline: 1.4207x; 1.0765x over previous
import functools

import jax
import jax.numpy as jnp
from jax import lax
from jax.experimental import pallas as pl
from jax.experimental.pallas import tpu as pltpu
from jax.experimental.pallas import tpu_sc as plsc

VOCAB = 1000000
D = 64
LMAX = 200
B = 4096
HID = 256
NFEAT = 8
LPAD = 256

CHUNKS = ((0, 128), (128, 72))


def _sc_pooled_sum():
    info = plsc.get_sparse_core_info()
    nc, ns = info.num_cores, info.num_subcores
    nw = nc * ns
    b_per_w = B // nw
    mesh = plsc.VectorSubcoreMesh(core_axis_name="c", subcore_axis_name="s")

    @functools.partial(
        pl.kernel,
        mesh=mesh,
        compiler_params=pltpu.CompilerParams(use_tc_tiling_on_sc=False),
        out_type=jax.ShapeDtypeStruct((B, D), jnp.float32),
        scratch_types=[
            pltpu.VMEM((LMAX,), jnp.int32),
            pltpu.VMEM((LMAX, D), jnp.float32),
            pltpu.VMEM((1, D), jnp.float32),
            pltpu.SemaphoreType.DMA,
        ],
    )
    def k(ids_hbm, table_hbm, out_hbm, idx_v, rows_v, acc_v, sem):
        wid = lax.axis_index("s") * nc + lax.axis_index("c")
        base = wid * b_per_w

        def body(i, _):
            b = base + i
            pltpu.sync_copy(ids_hbm.at[b], idx_v)
            copies = [
                pltpu.async_copy(
                    table_hbm.at[idx_v.at[pl.ds(off, n)]],
                    rows_v.at[pl.ds(off, n)],
                    sem,
                )
                for off, n in CHUNKS
            ]
            for c in copies:
                c.wait()
            for j in range(D // 16):
                def add(s, acc):
                    l = s * 4
                    for u in range(4):
                        acc = acc + rows_v[l + u, pl.ds(j * 16, 16)]
                    return acc
                acc = lax.fori_loop(0, LMAX // 4, add, jnp.zeros((16,), jnp.float32))
                acc_v[0, pl.ds(j * 16, 16)] = acc
            pltpu.sync_copy(acc_v, out_hbm.at[pl.ds(b, 1)])
            return 0

        lax.fori_loop(0, b_per_w, body, 0)

    return k


def _tc_head(ids_ref, pooled_ref, len_ref, tTL_ref, pos_ref, g_ref, b_ref,
             w1_ref, b1_ref, w2_ref, b2_ref, out_ref):
    f32 = jnp.float32
    m = (ids_ref[...] != 0).astype(f32)
    count = jnp.sum(m, axis=1, keepdims=True)
    pos_sum = jnp.dot(m, pos_ref[...], preferred_element_type=f32)
    denom = jnp.maximum(count, 1.0)
    seq = (pooled_ref[...] + pos_sum) / denom

    t = tTL_ref[0, 0]
    T = tTL_ref[0, 1]
    L = tTL_ref[0, 2]
    lens = len_ref[...].astype(f32)
    one = jnp.ones_like(lens)
    gap = lens - L
    rem = (T - t) * one
    prog = (t / jnp.maximum(T, 1.0)) * one
    need = gap / jnp.maximum(rem, 1.0)
    len_ratio = lens / jnp.maximum(L, 1.0)
    gap_ratio = gap / jnp.maximum(lens, 1.0)
    rem_ratio = ((T - t) / jnp.maximum(T, 1.0)) * one
    tgt_ratio = (L / jnp.maximum(T, 1.0)) * one
    feats = jnp.concatenate(
        [gap, rem, prog, need, len_ratio, gap_ratio, rem_ratio, tgt_ratio],
        axis=1)

    nf = D + NFEAT
    pad = jnp.zeros((seq.shape[0], 128 - nf), f32)
    fused = jnp.concatenate([seq, feats, pad], axis=1)
    mu = jnp.sum(fused, axis=1, keepdims=True) / nf
    var = jnp.sum(fused * fused, axis=1, keepdims=True) / nf - mu * mu
    ln = (fused - mu) * lax.rsqrt(var + 1e-5) * g_ref[...] + b_ref[...]

    h = jnp.dot(ln, w1_ref[...], preferred_element_type=f32) + b1_ref[...]
    h = 0.5 * h * (1.0 + lax.erf(h * 0.7071067811865476))
    logit = jnp.sum(h * w2_ref[...], axis=1, keepdims=True) + b2_ref[...]
    out_ref[...] = jax.nn.sigmoid(logit)


def kernel(token_ids, lengths, t, T, L, token_table, pos_table, ln_g, ln_b,
           W1, b1, W2, b2):
    f32 = jnp.float32
    ids = token_ids.astype(jnp.int32)

    pooled = _sc_pooled_sum()(ids, token_table)

    ids_pad = jnp.pad(ids, ((0, 0), (0, LPAD - LMAX)))
    pos_pad = jnp.pad(pos_table[:LMAX], ((0, LPAD - LMAX), (0, 0)))
    nf = D + NFEAT
    g_pad = jnp.pad(ln_g, (0, 128 - nf)).reshape(1, 128)
    b_pad = jnp.pad(ln_b, (0, 128 - nf)).reshape(1, 128)
    w1_pad = jnp.pad(W1, ((0, 128 - nf), (0, 0)))
    tTL = jnp.stack([jnp.asarray(t, f32), jnp.asarray(T, f32),
                     jnp.asarray(L, f32)]).reshape(1, 3)

    BLK = 512
    grid = (B // BLK,)
    rep = lambda i: (0, 0)
    out = pl.pallas_call(
        _tc_head,
        grid=grid,
        in_specs=[
            pl.BlockSpec((BLK, LPAD), lambda i: (i, 0)),
            pl.BlockSpec((BLK, D), lambda i: (i, 0)),
            pl.BlockSpec((BLK, 1), lambda i: (i, 0)),
            pl.BlockSpec(memory_space=pltpu.SMEM),
            pl.BlockSpec((LPAD, D), rep),
            pl.BlockSpec((1, 128), rep),
            pl.BlockSpec((1, 128), rep),
            pl.BlockSpec((128, HID), rep),
            pl.BlockSpec((1, HID), rep),
            pl.BlockSpec((1, HID), rep),
            pl.BlockSpec((1, 1), rep),
        ],
        out_specs=pl.BlockSpec((BLK, 1), lambda i: (i, 0)),
        out_shape=jax.ShapeDtypeStruct((B, 1), f32),
    )(ids_pad, pooled, lengths.astype(jnp.int32).reshape(B, 1), tTL, pos_pad,
      g_pad, b_pad, w1_pad, b1.reshape(1, HID), W2.reshape(1, HID),
      b2.reshape(1, 1))
    return out.reshape(B)


# speedup vs baseline: 1.5133x; 1.0652x over previous
import functools

import jax
import jax.numpy as jnp
from jax import lax
from jax.experimental import pallas as pl
from jax.experimental.pallas import tpu as pltpu
from jax.experimental.pallas import tpu_sc as plsc

VOCAB = 1000000
D = 64
LMAX = 200
B = 4096
HID = 256
NFEAT = 8
LPAD = 256

CHUNKS = ((0, 128), (128, 72))


def _sc_pooled_sum():
    info = plsc.get_sparse_core_info()
    nc, ns = info.num_cores, info.num_subcores
    nw = nc * ns
    b_per_w = B // nw
    mesh = plsc.VectorSubcoreMesh(core_axis_name="c", subcore_axis_name="s")

    @functools.partial(
        pl.kernel,
        mesh=mesh,
        compiler_params=pltpu.CompilerParams(use_tc_tiling_on_sc=False),
        out_type=jax.ShapeDtypeStruct((B, D), jnp.float32),
        scratch_types=[
            pltpu.VMEM((LMAX,), jnp.int32),
            pltpu.VMEM((LMAX, D), jnp.float32),
            pltpu.VMEM((1, D), jnp.float32),
            pltpu.SemaphoreType.DMA,
        ],
    )
    def k(ids_hbm, table_hbm, out_hbm, idx_v, rows_v, acc_v, sem):
        wid = lax.axis_index("s") * nc + lax.axis_index("c")
        base = wid * b_per_w

        def body(i, _):
            b = base + i
            pltpu.sync_copy(ids_hbm.at[b], idx_v)
            copies = [
                pltpu.async_copy(
                    table_hbm.at[idx_v.at[pl.ds(off, n)]],
                    rows_v.at[pl.ds(off, n)],
                    sem,
                )
                for off, n in CHUNKS
            ]
            for c in copies:
                c.wait()
            zero = jnp.zeros((16,), jnp.float32)

            def add(l, accs):
                return tuple(
                    accs[j] + rows_v[l, pl.ds(j * 16, 16)] for j in range(4)
                )

            accs = lax.fori_loop(0, LMAX, add, (zero,) * 4, unroll=4)
            for j in range(4):
                acc_v[0, pl.ds(j * 16, 16)] = accs[j]
            pltpu.sync_copy(acc_v, out_hbm.at[pl.ds(b, 1)])
            return 0

        lax.fori_loop(0, b_per_w, body, 0)

    return k


def _tc_head(ids_ref, pooled_ref, len_ref, tTL_ref, pos_ref, g_ref, b_ref,
             w1_ref, b1_ref, w2_ref, b2_ref, out_ref):
    f32 = jnp.float32
    m = (ids_ref[...] != 0).astype(f32)
    count = jnp.sum(m, axis=1, keepdims=True)
    pos_sum = jnp.dot(m, pos_ref[...], preferred_element_type=f32)
    denom = jnp.maximum(count, 1.0)
    seq = (pooled_ref[...] + pos_sum) / denom

    t = tTL_ref[0, 0]
    T = tTL_ref[0, 1]
    L = tTL_ref[0, 2]
    lens = len_ref[...].astype(f32)
    one = jnp.ones_like(lens)
    gap = lens - L
    rem = (T - t) * one
    prog = (t / jnp.maximum(T, 1.0)) * one
    need = gap / jnp.maximum(rem, 1.0)
    len_ratio = lens / jnp.maximum(L, 1.0)
    gap_ratio = gap / jnp.maximum(lens, 1.0)
    rem_ratio = ((T - t) / jnp.maximum(T, 1.0)) * one
    tgt_ratio = (L / jnp.maximum(T, 1.0)) * one
    feats = jnp.concatenate(
        [gap, rem, prog, need, len_ratio, gap_ratio, rem_ratio, tgt_ratio],
        axis=1)

    nf = D + NFEAT
    pad = jnp.zeros((seq.shape[0], 128 - nf), f32)
    fused = jnp.concatenate([seq, feats, pad], axis=1)
    mu = jnp.sum(fused, axis=1, keepdims=True) / nf
    var = jnp.sum(fused * fused, axis=1, keepdims=True) / nf - mu * mu
    ln = (fused - mu) * lax.rsqrt(var + 1e-5) * g_ref[...] + b_ref[...]

    h = jnp.dot(ln, w1_ref[...], preferred_element_type=f32) + b1_ref[...]
    h = 0.5 * h * (1.0 + lax.erf(h * 0.7071067811865476))
    logit = jnp.sum(h * w2_ref[...], axis=1, keepdims=True) + b2_ref[...]
    out_ref[...] = jax.nn.sigmoid(logit)


def kernel(token_ids, lengths, t, T, L, token_table, pos_table, ln_g, ln_b,
           W1, b1, W2, b2):
    f32 = jnp.float32
    ids = token_ids.astype(jnp.int32)

    pooled = _sc_pooled_sum()(ids, token_table)

    ids_pad = jnp.pad(ids, ((0, 0), (0, LPAD - LMAX)))
    pos_pad = jnp.pad(pos_table[:LMAX], ((0, LPAD - LMAX), (0, 0)))
    nf = D + NFEAT
    g_pad = jnp.pad(ln_g, (0, 128 - nf)).reshape(1, 128)
    b_pad = jnp.pad(ln_b, (0, 128 - nf)).reshape(1, 128)
    w1_pad = jnp.pad(W1, ((0, 128 - nf), (0, 0)))
    tTL = jnp.stack([jnp.asarray(t, f32), jnp.asarray(T, f32),
                     jnp.asarray(L, f32)]).reshape(1, 3)

    BLK = 512
    grid = (B // BLK,)
    rep = lambda i: (0, 0)
    out = pl.pallas_call(
        _tc_head,
        grid=grid,
        in_specs=[
            pl.BlockSpec((BLK, LPAD), lambda i: (i, 0)),
            pl.BlockSpec((BLK, D), lambda i: (i, 0)),
            pl.BlockSpec((BLK, 1), lambda i: (i, 0)),
            pl.BlockSpec(memory_space=pltpu.SMEM),
            pl.BlockSpec((LPAD, D), rep),
            pl.BlockSpec((1, 128), rep),
            pl.BlockSpec((1, 128), rep),
            pl.BlockSpec((128, HID), rep),
            pl.BlockSpec((1, HID), rep),
            pl.BlockSpec((1, HID), rep),
            pl.BlockSpec((1, 1), rep),
        ],
        out_specs=pl.BlockSpec((BLK, 1), lambda i: (i, 0)),
        out_shape=jax.ShapeDtypeStruct((B, 1), f32),
    )(ids_pad, pooled, lengths.astype(jnp.int32).reshape(B, 1), tTL, pos_pad,
      g_pad, b_pad, w1_pad, b1.reshape(1, HID), W2.reshape(1, HID),
      b2.reshape(1, 1))
    return out.reshape(B)


# speedup vs baseline: 1.6920x; 1.1180x over previous
import functools

import jax
import jax.numpy as jnp
from jax import lax
from jax.experimental import pallas as pl
from jax.experimental.pallas import tpu as pltpu
from jax.experimental.pallas import tpu_sc as plsc

VOCAB = 1000000
D = 64
LMAX = 200
B = 4096
HID = 256
NFEAT = 8
LPAD = 256

CHUNKS = ((0, 128), (128, 72))


def _sc_pooled_sum():
    info = plsc.get_sparse_core_info()
    nc, ns = info.num_cores, info.num_subcores
    nw = nc * ns
    b_per_w = B // nw
    mesh = plsc.VectorSubcoreMesh(core_axis_name="c", subcore_axis_name="s")

    @functools.partial(
        pl.kernel,
        mesh=mesh,
        compiler_params=pltpu.CompilerParams(use_tc_tiling_on_sc=False),
        out_type=jax.ShapeDtypeStruct((B, D), jnp.float32),
        scratch_types=[
            pltpu.VMEM((2, LMAX), jnp.int32),
            pltpu.VMEM((2, LMAX, D), jnp.float32),
            pltpu.VMEM((2, 1, D), jnp.float32),
            pltpu.SemaphoreType.DMA,
            pltpu.SemaphoreType.DMA,
        ],
    )
    def k(ids_hbm, table_hbm, out_hbm, idx_v, rows_v, acc_v, sem0, sem1):
        wid = lax.axis_index("s") * nc + lax.axis_index("c")
        base = wid * b_per_w
        sems = (sem0, sem1)

        def fetch(b, s):
            pltpu.sync_copy(ids_hbm.at[b], idx_v.at[s])
            return [
                pltpu.async_copy(
                    table_hbm.at[idx_v.at[s, pl.ds(off, n)]],
                    rows_v.at[s, pl.ds(off, n)],
                    sems[s],
                )
                for off, n in CHUNKS
            ]

        def accum(b, s, copies):
            for c in copies:
                c.wait()
            zero = jnp.zeros((16,), jnp.float32)

            def add(l, accs):
                return tuple(
                    accs[j] + rows_v[s, l, pl.ds(j * 16, 16)]
                    for j in range(4)
                )

            accs = lax.fori_loop(0, LMAX, add, (zero,) * 4, unroll=4)
            for j in range(4):
                acc_v[s, 0, pl.ds(j * 16, 16)] = accs[j]
            pltpu.sync_copy(acc_v.at[s], out_hbm.at[pl.ds(b, 1)])

        def body(p, _):
            q = base + 2 * p
            c0 = fetch(q, 0)
            c1 = fetch(q + 1, 1)
            accum(q, 0, c0)
            accum(q + 1, 1, c1)
            return 0

        lax.fori_loop(0, b_per_w // 2, body, 0)

    return k


def _tc_head(ids_ref, pooled_ref, len_ref, tTL_ref, pos_ref, g_ref, b_ref,
             w1_ref, b1_ref, w2_ref, b2_ref, out_ref):
    f32 = jnp.float32
    m = (ids_ref[...] != 0).astype(f32)
    count = jnp.sum(m, axis=1, keepdims=True)
    pos_sum = jnp.dot(m, pos_ref[...], preferred_element_type=f32)
    denom = jnp.maximum(count, 1.0)
    seq = (pooled_ref[...] + pos_sum) / denom

    t = tTL_ref[0, 0]
    T = tTL_ref[0, 1]
    L = tTL_ref[0, 2]
    lens = len_ref[...].astype(f32)
    one = jnp.ones_like(lens)
    gap = lens - L
    rem = (T - t) * one
    prog = (t / jnp.maximum(T, 1.0)) * one
    need = gap / jnp.maximum(rem, 1.0)
    len_ratio = lens / jnp.maximum(L, 1.0)
    gap_ratio = gap / jnp.maximum(lens, 1.0)
    rem_ratio = ((T - t) / jnp.maximum(T, 1.0)) * one
    tgt_ratio = (L / jnp.maximum(T, 1.0)) * one
    feats = jnp.concatenate(
        [gap, rem, prog, need, len_ratio, gap_ratio, rem_ratio, tgt_ratio],
        axis=1)

    nf = D + NFEAT
    pad = jnp.zeros((seq.shape[0], 128 - nf), f32)
    fused = jnp.concatenate([seq, feats, pad], axis=1)
    mu = jnp.sum(fused, axis=1, keepdims=True) / nf
    var = jnp.sum(fused * fused, axis=1, keepdims=True) / nf - mu * mu
    ln = (fused - mu) * lax.rsqrt(var + 1e-5) * g_ref[...] + b_ref[...]

    h = jnp.dot(ln, w1_ref[...], preferred_element_type=f32) + b1_ref[...]
    h = 0.5 * h * (1.0 + lax.erf(h * 0.7071067811865476))
    logit = jnp.sum(h * w2_ref[...], axis=1, keepdims=True) + b2_ref[...]
    out_ref[...] = jax.nn.sigmoid(logit)


def kernel(token_ids, lengths, t, T, L, token_table, pos_table, ln_g, ln_b,
           W1, b1, W2, b2):
    f32 = jnp.float32
    ids = token_ids.astype(jnp.int32)

    pooled = _sc_pooled_sum()(ids, token_table)

    ids_pad = jnp.pad(ids, ((0, 0), (0, LPAD - LMAX)))
    pos_pad = jnp.pad(pos_table[:LMAX], ((0, LPAD - LMAX), (0, 0)))
    nf = D + NFEAT
    g_pad = jnp.pad(ln_g, (0, 128 - nf)).reshape(1, 128)
    b_pad = jnp.pad(ln_b, (0, 128 - nf)).reshape(1, 128)
    w1_pad = jnp.pad(W1, ((0, 128 - nf), (0, 0)))
    tTL = jnp.stack([jnp.asarray(t, f32), jnp.asarray(T, f32),
                     jnp.asarray(L, f32)]).reshape(1, 3)

    BLK = 512
    grid = (B // BLK,)
    rep = lambda i: (0, 0)
    out = pl.pallas_call(
        _tc_head,
        grid=grid,
        in_specs=[
            pl.BlockSpec((BLK, LPAD), lambda i: (i, 0)),
            pl.BlockSpec((BLK, D), lambda i: (i, 0)),
            pl.BlockSpec((BLK, 1), lambda i: (i, 0)),
            pl.BlockSpec(memory_space=pltpu.SMEM),
            pl.BlockSpec((LPAD, D), rep),
            pl.BlockSpec((1, 128), rep),
            pl.BlockSpec((1, 128), rep),
            pl.BlockSpec((128, HID), rep),
            pl.BlockSpec((1, HID), rep),
            pl.BlockSpec((1, HID), rep),
            pl.BlockSpec((1, 1), rep),
        ],
        out_specs=pl.BlockSpec((BLK, 1), lambda i: (i, 0)),
        out_shape=jax.ShapeDtypeStruct((B, 1), f32),
    )(ids_pad, pooled, lengths.astype(jnp.int32).reshape(B, 1), tTL, pos_pad,
      g_pad, b_pad, w1_pad, b1.reshape(1, HID), W2.reshape(1, HID),
      b2.reshape(1, 1))
    return out.reshape(B)
